# baseline (speedup 1.0000x reference)
import functools

import jax
import jax.numpy as jnp
from jax import lax
from jax.experimental import pallas as pl
from jax.experimental.pallas import tpu as pltpu
from jax.experimental.pallas import tpu_sc as plsc

N = 10000
E = 320000
D = 128
L = 4
NUM_GRAPHS = 64

NC = 2
NS = 16
NW = NC * NS
EP = E // NW
CHUNK = 64
CH = 160
EPP = CH * CHUNK
PP = EPP - EP
NDISCARD = 112
NPAD = N + NDISCARD
ROWS_PT = 632
OUT_ROWS = NS * ROWS_PT
PSHIFT = 15
PMASK = (1 << PSHIFT) - 1
NRING = 4

BS = 2000
NBLK = N // BS

_F32 = jnp.float32


def _sc_body(p_hbm, hm_hbm, tbl_hbm, out_hbm,
             pring, sring, dring, ering, gasb, gbuf, rbuf, obuf, agg,
             rsem0, rsem1, asem0, asem1, ssem0, ssem1, isem0, isem1):
  c = lax.axis_index("c")
  s = lax.axis_index("s")
  wid = c * NS + s

  def zrow(r, carry):
    for k in range(8):
      rbuf[0, r, pl.ds(k * 16, 16)] = jnp.zeros((16,), _F32)
    return carry
  lax.fori_loop(0, CHUNK, zrow, 0)
  base = s * ROWS_PT
  for t in range(ROWS_PT // CHUNK):
    pltpu.sync_copy(rbuf.at[0], agg.at[pl.ds(base + t * CHUNK, CHUNK)])
  rem = ROWS_PT % CHUNK
  pltpu.sync_copy(rbuf.at[0, pl.ds(0, rem)],
                  agg.at[pl.ds(base + ROWS_PT - rem, rem)])
  plsc.subcore_barrier()

  rsems = (rsem0, rsem1)
  asems = (asem0, asem1)
  ssems = (ssem0, ssem1)
  isems = (isem0, isem1)

  def start_idx4(j, islot):
    pltpu.async_copy(p_hbm.at[wid, pl.ds(j, 4)], pring.at[islot],
                     isems[islot])

  def wait_idx4(islot):
    pltpu.make_async_copy(p_hbm.at[wid, pl.ds(0, 4)], pring.at[islot],
                          isems[islot]).wait()

  def unpack(islot, sub, slot):
    for q in range(CHUNK // 16):
      pv = pring[islot, sub, pl.ds(q * 16, 16)]
      sv = lax.shift_right_logical(pv, PSHIFT)
      dv = lax.bitwise_and(pv, PMASK)
      sring[slot, pl.ds(q * 16, 16)] = sv
      dring[slot, pl.ds(q * 16, 16)] = dv
      ering[slot, pl.ds(q * 16, 16)] = sv + sv
      ering[slot, pl.ds(CHUNK + q * 16, 16)] = dv + dv + jnp.ones((16,), jnp.int32)

  def start_gathers(slot, b):
    pltpu.async_copy(hm_hbm.at[sring.at[slot]], rbuf.at[b], rsems[b])
    pltpu.async_copy(tbl_hbm.at[ering.at[slot]], gasb.at[b], asems[b])

  def wait_gathers(b):
    pltpu.make_async_copy(hm_hbm.at[sring.at[0]], rbuf.at[b], rsems[b]).wait()
    pltpu.make_async_copy(tbl_hbm.at[ering.at[0]], gasb.at[b], asems[b]).wait()

  def start_scatter(slot, b):
    pltpu.async_copy(obuf.at[b], agg.at[dring.at[slot]], ssems[b], add=True)

  def wait_scatter(b):
    pltpu.make_async_copy(obuf.at[b], agg.at[dring.at[0]], ssems[b]).wait()

  def compute(b):
    for q in range(CHUNK // 16):
      a = gasb[b, pl.ds(q * 16, 16)] + gasb[b, pl.ds(CHUNK + q * 16, 16)]
      gbuf[pl.ds(b * CHUNK + q * 16, 16)] = 1.0 / (1.0 + jnp.exp(-a))

    @plsc.parallel_loop(0, CHUNK, 1, unroll=4)
    def _(r):
      gb = plsc.load_gather(gbuf, [jnp.full((16,), b * CHUNK, jnp.int32) + r])
      for k in range(8):
        obuf[b, r, pl.ds(k * 16, 16)] = gb * rbuf[b, r, pl.ds(k * 16, 16)]

  start_idx4(0, 0)
  start_idx4(4, 1)
  wait_idx4(0)
  unpack(0, 0, 0)
  unpack(0, 1, 1)
  start_gathers(0, 0)
  start_gathers(1, 1)

  def body(g2, carry):
    j0 = 8 * g2
    for b in range(8):
      j = j0 + b
      b2 = b % 2
      wait_gathers(b2)

      @pl.when(j >= 2)
      def _():
        wait_scatter(b2)

      if b == 2:
        wait_idx4(1)

        @pl.when(j0 + 8 < CH)
        def _():
          start_idx4(j0 + 8, 0)
      if b == 6:
        @pl.when(j0 + 8 < CH)
        def _():
          wait_idx4(0)

        @pl.when(j0 + 12 < CH)
        def _():
          start_idx4(j0 + 12, 1)

      @pl.when(j + 2 < CH)
      def _():
        unpack(((b + 2) // 4) % 2, (b + 2) % 4, (b + 2) % 4)

      compute(b2)
      start_scatter(b % 4, b2)

      @pl.when(j + 2 < CH)
      def _():
        start_gathers((b + 2) % 4, b2)
    return carry

  lax.fori_loop(0, CH // 8, body, 0)
  wait_scatter(0)
  wait_scatter(1)
  plsc.subcore_barrier()
  pltpu.sync_copy(agg.at[pl.ds(base, ROWS_PT)],
                  out_hbm.at[c, pl.ds(base, ROWS_PT)])


_sc_edge_pass = functools.partial(
    pl.kernel,
    out_type=jax.ShapeDtypeStruct((NC, OUT_ROWS, D), _F32),
    mesh=plsc.VectorSubcoreMesh(core_axis_name="c", subcore_axis_name="s",
                                num_cores=NC, num_subcores=NS),
    scratch_types=[
        pltpu.VMEM((2, 4, CHUNK), jnp.int32),
        pltpu.VMEM((NRING, CHUNK), jnp.int32),
        pltpu.VMEM((NRING, CHUNK), jnp.int32),
        pltpu.VMEM((NRING, 2 * CHUNK), jnp.int32),
        pltpu.VMEM((2, 2 * CHUNK), _F32),
        pltpu.VMEM((2 * CHUNK,), _F32),
        pltpu.VMEM((2, CHUNK, D), _F32),
        pltpu.VMEM((2, CHUNK, D), _F32),
        pltpu.VMEM_SHARED((NPAD, D), _F32),
    ] + [pltpu.SemaphoreType.DMA] * 8,
    compiler_params=pltpu.CompilerParams(needs_layout_passes=False),
)(_sc_body)


def _produce(h, wm_ref, bm_ref, wr_ref, br_ref, wa_ref, bac_ref,
             hm_ref, tb_ref, hr_ref):
  hm_ref[...] = jnp.dot(h, wm_ref[...], preferred_element_type=_F32) + bm_ref[...]
  hr_ref[...] = jnp.dot(h, wr_ref[...], preferred_element_type=_F32) + br_ref[...]
  tb_ref[...] = jnp.dot(h, wa_ref[...], preferred_element_type=_F32) + bac_ref[...]


def _k_in_body(x_ref, win_ref, bin_ref, wm_ref, bm_ref, wr_ref, br_ref,
               wa_ref, bac_ref, hm_ref, tb_ref, hr_ref):
  h = jnp.maximum(
      jnp.dot(x_ref[...], win_ref[...], preferred_element_type=_F32)
      + bin_ref[...], 0.0)
  _produce(h, wm_ref, bm_ref, wr_ref, br_ref, wa_ref, bac_ref,
           hm_ref, tb_ref, hr_ref)


def _k_stats_body(aggp_ref, hr_ref, z_ref, st_ref):
  i = pl.program_id(0)
  zb = aggp_ref[0] + aggp_ref[1] + hr_ref[...]
  z_ref[...] = zb

  @pl.when(i == 0)
  def _():
    st_ref[...] = jnp.zeros_like(st_ref)

  colsum = jnp.sum(zb, axis=0, keepdims=True)
  colsq = jnp.sum(zb * zb, axis=0, keepdims=True)
  upd = jnp.concatenate([colsum, colsq, jnp.zeros((6, D), _F32)], axis=0)
  st_ref[...] = st_ref[...] + upd


def _bn_relu(z_ref, st_ref, gamma_ref, beta_ref):
  stt = st_ref[...]
  mean = stt[0:1, :] / N
  var = stt[1:2, :] / N - mean * mean
  inv = lax.rsqrt(var + 1e-5)
  return jnp.maximum((z_ref[...] - mean) * (inv * gamma_ref[...])
                     + beta_ref[...], 0.0)


def _k_next_body(z_ref, st_ref, gamma_ref, beta_ref, wm_ref, bm_ref,
                 wr_ref, br_ref, wa_ref, bac_ref, hm_ref, tb_ref, hr_ref):
  h = _bn_relu(z_ref, st_ref, gamma_ref, beta_ref)
  _produce(h, wm_ref, bm_ref, wr_ref, br_ref, wa_ref, bac_ref,
           hm_ref, tb_ref, hr_ref)


def _k_final_body(z_ref, st_ref, gamma_ref, beta_ref, batch_ref,
                  w1_ref, b1_ref, w2_ref, b2_ref, out_ref, pooled_ref):
  i = pl.program_id(0)
  h = _bn_relu(z_ref, st_ref, gamma_ref, beta_ref)
  bb = batch_ref[0]
  gids = lax.broadcasted_iota(jnp.int32, (NUM_GRAPHS, BS), 0)
  onehot = jnp.where(gids == bb, 1.0, 0.0).astype(_F32)

  @pl.when(i == 0)
  def _():
    pooled_ref[...] = jnp.zeros_like(pooled_ref)

  pooled_ref[...] = pooled_ref[...] + jnp.dot(
      onehot, h, preferred_element_type=_F32)

  @pl.when(i == NBLK - 1)
  def _():
    p = pooled_ref[...]
    o1 = jnp.maximum(jnp.dot(p, w1_ref[...], preferred_element_type=_F32)
                     + b1_ref[...], 0.0)
    out_ref[...] = (jnp.dot(o1, w2_ref[...], preferred_element_type=_F32)
                    + b2_ref[...]) * 0.5


def _row_spec():
  return pl.BlockSpec((BS, D), lambda i: (i, 0))


def _full_spec(shape):
  return pl.BlockSpec(shape, lambda i: tuple(0 for _ in shape))


_k_in = pl.pallas_call(
    _k_in_body,
    grid=(NBLK,),
    in_specs=[
        _row_spec(),
        _full_spec((D, D)), _full_spec((1, D)),
        _full_spec((D, D)), _full_spec((1, D)),
        _full_spec((D, D)), _full_spec((1, D)),
        _full_spec((D, 2)), _full_spec((1, 2)),
    ],
    out_specs=[_row_spec(), pl.BlockSpec((BS, 2), lambda i: (i, 0)), _row_spec()],
    out_shape=[
        jax.ShapeDtypeStruct((N, D), _F32),
        jax.ShapeDtypeStruct((N, 2), _F32),
        jax.ShapeDtypeStruct((N, D), _F32),
    ],
)

_k_stats = pl.pallas_call(
    _k_stats_body,
    grid=(NBLK,),
    in_specs=[
        pl.BlockSpec((NC, BS, D), lambda i: (0, i, 0)),
        _row_spec(),
    ],
    out_specs=[_row_spec(), _full_spec((8, D))],
    out_shape=[
        jax.ShapeDtypeStruct((N, D), _F32),
        jax.ShapeDtypeStruct((8, D), _F32),
    ],
)

_k_next = pl.pallas_call(
    _k_next_body,
    grid=(NBLK,),
    in_specs=[
        _row_spec(),
        _full_spec((8, D)),
        _full_spec((1, D)), _full_spec((1, D)),
        _full_spec((D, D)), _full_spec((1, D)),
        _full_spec((D, D)), _full_spec((1, D)),
        _full_spec((D, 2)), _full_spec((1, 2)),
    ],
    out_specs=[_row_spec(), pl.BlockSpec((BS, 2), lambda i: (i, 0)), _row_spec()],
    out_shape=[
        jax.ShapeDtypeStruct((N, D), _F32),
        jax.ShapeDtypeStruct((N, 2), _F32),
        jax.ShapeDtypeStruct((N, D), _F32),
    ],
)

_k_final = pl.pallas_call(
    _k_final_body,
    grid=(NBLK,),
    in_specs=[
        _row_spec(),
        _full_spec((8, D)),
        _full_spec((1, D)), _full_spec((1, D)),
        pl.BlockSpec((1, 1, BS), lambda i: (i, 0, 0)),
        _full_spec((D, D // 2)), _full_spec((1, D // 2)),
        _full_spec((D // 2, 10)), _full_spec((1, 10)),
    ],
    out_specs=_full_spec((NUM_GRAPHS, 10)),
    out_shape=jax.ShapeDtypeStruct((NUM_GRAPHS, 10), _F32),
    scratch_shapes=[pltpu.VMEM((NUM_GRAPHS, D), _F32)],
)


def kernel(x, edge_index, batch, W_in, b_in, Wa, ba, Wm, bm, Wr, br,
           gamma, beta, W1, b1, W2, b2):
  src = edge_index[0].astype(jnp.int32)
  dst = edge_index[1].astype(jnp.int32)
  packed = src * (1 << PSHIFT) + dst
  pad_s = (jnp.arange(PP, dtype=jnp.int32) * 41) % N
  pad_d = N + (jnp.arange(PP, dtype=jnp.int32) % NDISCARD)
  pad_p = pad_s * (1 << PSHIFT) + pad_d
  p3 = jnp.concatenate(
      [packed.reshape(NW, EP), jnp.broadcast_to(pad_p, (NW, PP))],
      axis=1).reshape(NW, CH, CHUNK)
  batch3 = batch.astype(jnp.int32).reshape(NBLK, 1, BS)

  def wa2(i):
    return Wa[i, :, 0].reshape(2, D).transpose(1, 0)

  def bac(i):
    return jnp.concatenate([ba[i], jnp.zeros((1,), _F32)]).reshape(1, 2)

  def row(v):
    return v.reshape(1, -1)

  hm, tbl, hr = _k_in(x, W_in, row(b_in), Wm[0], row(bm[0]),
                      Wr[0], row(br[0]), wa2(0), bac(0))
  for i in range(L):
    tbl_i = jnp.pad(tbl, ((0, NDISCARD), (0, 0))).reshape(2 * NPAD)
    aggp = _sc_edge_pass(p3, hm, tbl_i)
    z, st = _k_stats(aggp, hr)
    if i < L - 1:
      hm, tbl, hr = _k_next(z, st, row(gamma[i]), row(beta[i]),
                            Wm[i + 1], row(bm[i + 1]),
                            Wr[i + 1], row(br[i + 1]),
                            wa2(i + 1), bac(i + 1))
  logits = _k_final(z, st, row(gamma[L - 1]), row(beta[L - 1]), batch3,
                    W1, row(b1), W2, row(b2))
  return logits


# speedup vs baseline: 1.1018x; 1.1018x over previous
import functools

import jax
import jax.numpy as jnp
from jax import lax
from jax.experimental import pallas as pl
from jax.experimental.pallas import tpu as pltpu
from jax.experimental.pallas import tpu_sc as plsc

N = 10000
E = 320000
D = 128
L = 4
NUM_GRAPHS = 64

NC = 2
NS = 16
NW = NC * NS
EP = E // NW
CHUNK = 64
NB = 3
CH = 162
EPP = CH * CHUNK
PP = EPP - EP
NDISCARD = 112
NPAD = N + NDISCARD
ROWS_PT = 632
OUT_ROWS = NS * ROWS_PT
PSHIFT = 15
PMASK = (1 << PSHIFT) - 1
NRING = 4

BS = 2000
NBLK = N // BS

_F32 = jnp.float32


def _sc_body(p_hbm, hm_hbm, gtab_hbm, out_hbm,
             pring, sring, dring, gtab, gbuf, rbuf, agg,
             rsem0, rsem1, rsem2, ssem0, ssem1, ssem2,
             isem0, isem1, isem2):
  c = lax.axis_index("c")
  s = lax.axis_index("s")
  wid = c * NS + s

  pltpu.sync_copy(gtab_hbm, gtab)

  def zrow(r, carry):
    for k in range(8):
      rbuf[0, r, pl.ds(k * 16, 16)] = jnp.zeros((16,), _F32)
    return carry
  lax.fori_loop(0, CHUNK, zrow, 0)
  base = s * ROWS_PT
  for tt in range(ROWS_PT // CHUNK):
    pltpu.sync_copy(rbuf.at[0], agg.at[pl.ds(base + tt * CHUNK, CHUNK)])
  rem = ROWS_PT % CHUNK
  pltpu.sync_copy(rbuf.at[0, pl.ds(0, rem)],
                  agg.at[pl.ds(base + ROWS_PT - rem, rem)])
  plsc.subcore_barrier()

  rsems = (rsem0, rsem1, rsem2)
  ssems = (ssem0, ssem1, ssem2)
  isems = (isem0, isem1, isem2)

  def start_idx(j, b):
    pltpu.async_copy(p_hbm.at[wid, j], pring.at[b], isems[b])

  def wait_idx(b):
    pltpu.make_async_copy(p_hbm.at[wid, 0], pring.at[b], isems[b]).wait()

  def unpack(b):
    for q in range(CHUNK // 16):
      pv = pring[b, pl.ds(q * 16, 16)]
      sring[b, pl.ds(q * 16, 16)] = lax.shift_right_logical(pv, PSHIFT)
      dring[b, pl.ds(q * 16, 16)] = lax.bitwise_and(pv, PMASK)

  def start_gather(b):
    pltpu.async_copy(hm_hbm.at[sring.at[b]], rbuf.at[b], rsems[b])

  def wait_gather(b):
    pltpu.make_async_copy(hm_hbm.at[sring.at[0]], rbuf.at[b], rsems[b]).wait()

  def start_scatter(b):
    pltpu.async_copy(rbuf.at[b], agg.at[dring.at[b]], ssems[b], add=True)

  def wait_scatter(b):
    pltpu.make_async_copy(rbuf.at[b], agg.at[dring.at[0]], ssems[b]).wait()

  himask = jnp.full((16,), -65536, jnp.int32)

  def compute(b):
    for q in range(CHUNK // 16):
      sv = sring[b, pl.ds(q * 16, 16)]
      dv = dring[b, pl.ds(q * 16, 16)]
      ws = plsc.load_gather(gtab, [sv])
      wd = plsc.load_gather(gtab, [dv])
      asf = plsc.bitcast(lax.shift_left(ws, 16), _F32)
      adf = plsc.bitcast(lax.bitwise_and(wd, himask), _F32)
      a = asf + adf
      gbuf[pl.ds(b * CHUNK + q * 16, 16)] = 1.0 / (1.0 + jnp.exp(-a))

    @plsc.parallel_loop(0, CHUNK, 1, unroll=4)
    def _(r):
      gb = plsc.load_gather(gbuf, [jnp.full((16,), b * CHUNK, jnp.int32) + r])
      for k in range(8):
        rbuf[b, r, pl.ds(k * 16, 16)] = gb * rbuf[b, r, pl.ds(k * 16, 16)]

  for j in range(NB):
    start_idx(j, j)
  for j in range(2):
    wait_idx(j)
    unpack(j)
    start_gather(j)

  def body(g, carry):
    for b in range(NB):
      j = NB * g + b
      wait_gather(b)
      compute(b)
      start_scatter(b)

      @pl.when(j + NB < CH)
      def _():
        start_idx(j + NB, b)

      @pl.when((j >= 1) & (j + 2 < CH))
      def _():
        wait_scatter((b + 2) % NB)

      @pl.when(j + 2 < CH)
      def _():
        wait_idx((b + 2) % NB)

      @pl.when(j + 2 < CH)
      def _():
        unpack((b + 2) % NB)

      @pl.when(j + 2 < CH)
      def _():
        start_gather((b + 2) % NB)
    return carry

  lax.fori_loop(0, CH // NB, body, 0)
  wait_scatter(0)
  wait_scatter(1)
  wait_scatter(2)
  plsc.subcore_barrier()
  pltpu.sync_copy(agg.at[pl.ds(base, ROWS_PT)],
                  out_hbm.at[c, pl.ds(base, ROWS_PT)])


_sc_edge_pass = functools.partial(
    pl.kernel,
    out_type=jax.ShapeDtypeStruct((NC, OUT_ROWS, D), _F32),
    mesh=plsc.VectorSubcoreMesh(core_axis_name="c", subcore_axis_name="s",
                                num_cores=NC, num_subcores=NS),
    scratch_types=[
        pltpu.VMEM((NB, CHUNK), jnp.int32),
        pltpu.VMEM((NB, CHUNK), jnp.int32),
        pltpu.VMEM((NB, CHUNK), jnp.int32),
        pltpu.VMEM((NPAD,), jnp.int32),
        pltpu.VMEM((NB * CHUNK,), _F32),
        pltpu.VMEM((NB, CHUNK, D), _F32),
        pltpu.VMEM_SHARED((NPAD, D), _F32),
    ] + [pltpu.SemaphoreType.DMA] * 9,
    compiler_params=pltpu.CompilerParams(needs_layout_passes=False),
)(_sc_body)


def _produce(h, wm_ref, bm_ref, wr_ref, br_ref, wa_ref, bac_ref,
             hm_ref, tb_ref, hr_ref):
  hm_ref[...] = jnp.dot(h, wm_ref[...], preferred_element_type=_F32) + bm_ref[...]
  hr_ref[...] = jnp.dot(h, wr_ref[...], preferred_element_type=_F32) + br_ref[...]
  tb_ref[...] = jnp.dot(h, wa_ref[...], preferred_element_type=_F32) + bac_ref[...]


def _k_in_body(x_ref, win_ref, bin_ref, wm_ref, bm_ref, wr_ref, br_ref,
               wa_ref, bac_ref, hm_ref, tb_ref, hr_ref):
  h = jnp.maximum(
      jnp.dot(x_ref[...], win_ref[...], preferred_element_type=_F32)
      + bin_ref[...], 0.0)
  _produce(h, wm_ref, bm_ref, wr_ref, br_ref, wa_ref, bac_ref,
           hm_ref, tb_ref, hr_ref)


def _k_stats_body(aggp_ref, hr_ref, z_ref, st_ref):
  i = pl.program_id(0)
  zb = aggp_ref[0] + aggp_ref[1] + hr_ref[...]
  z_ref[...] = zb

  @pl.when(i == 0)
  def _():
    st_ref[...] = jnp.zeros_like(st_ref)

  colsum = jnp.sum(zb, axis=0, keepdims=True)
  colsq = jnp.sum(zb * zb, axis=0, keepdims=True)
  upd = jnp.concatenate([colsum, colsq, jnp.zeros((6, D), _F32)], axis=0)
  st_ref[...] = st_ref[...] + upd


def _bn_relu(z_ref, st_ref, gamma_ref, beta_ref):
  stt = st_ref[...]
  mean = stt[0:1, :] / N
  var = stt[1:2, :] / N - mean * mean
  inv = lax.rsqrt(var + 1e-5)
  return jnp.maximum((z_ref[...] - mean) * (inv * gamma_ref[...])
                     + beta_ref[...], 0.0)


def _k_next_body(z_ref, st_ref, gamma_ref, beta_ref, wm_ref, bm_ref,
                 wr_ref, br_ref, wa_ref, bac_ref, hm_ref, tb_ref, hr_ref):
  h = _bn_relu(z_ref, st_ref, gamma_ref, beta_ref)
  _produce(h, wm_ref, bm_ref, wr_ref, br_ref, wa_ref, bac_ref,
           hm_ref, tb_ref, hr_ref)


def _k_final_body(z_ref, st_ref, gamma_ref, beta_ref, batch_ref,
                  w1_ref, b1_ref, w2_ref, b2_ref, out_ref, pooled_ref):
  i = pl.program_id(0)
  h = _bn_relu(z_ref, st_ref, gamma_ref, beta_ref)
  bb = batch_ref[0]
  gids = lax.broadcasted_iota(jnp.int32, (NUM_GRAPHS, BS), 0)
  onehot = jnp.where(gids == bb, 1.0, 0.0).astype(_F32)

  @pl.when(i == 0)
  def _():
    pooled_ref[...] = jnp.zeros_like(pooled_ref)

  pooled_ref[...] = pooled_ref[...] + jnp.dot(
      onehot, h, preferred_element_type=_F32)

  @pl.when(i == NBLK - 1)
  def _():
    p = pooled_ref[...]
    o1 = jnp.maximum(jnp.dot(p, w1_ref[...], preferred_element_type=_F32)
                     + b1_ref[...], 0.0)
    out_ref[...] = (jnp.dot(o1, w2_ref[...], preferred_element_type=_F32)
                    + b2_ref[...]) * 0.5


def _row_spec():
  return pl.BlockSpec((BS, D), lambda i: (i, 0))


def _full_spec(shape):
  return pl.BlockSpec(shape, lambda i: tuple(0 for _ in shape))


_k_in = pl.pallas_call(
    _k_in_body,
    grid=(NBLK,),
    in_specs=[
        _row_spec(),
        _full_spec((D, D)), _full_spec((1, D)),
        _full_spec((D, D)), _full_spec((1, D)),
        _full_spec((D, D)), _full_spec((1, D)),
        _full_spec((D, 2)), _full_spec((1, 2)),
    ],
    out_specs=[_row_spec(), pl.BlockSpec((BS, 2), lambda i: (i, 0)), _row_spec()],
    out_shape=[
        jax.ShapeDtypeStruct((N, D), _F32),
        jax.ShapeDtypeStruct((N, 2), _F32),
        jax.ShapeDtypeStruct((N, D), _F32),
    ],
)

_k_stats = pl.pallas_call(
    _k_stats_body,
    grid=(NBLK,),
    in_specs=[
        pl.BlockSpec((NC, BS, D), lambda i: (0, i, 0)),
        _row_spec(),
    ],
    out_specs=[_row_spec(), _full_spec((8, D))],
    out_shape=[
        jax.ShapeDtypeStruct((N, D), _F32),
        jax.ShapeDtypeStruct((8, D), _F32),
    ],
)

_k_next = pl.pallas_call(
    _k_next_body,
    grid=(NBLK,),
    in_specs=[
        _row_spec(),
        _full_spec((8, D)),
        _full_spec((1, D)), _full_spec((1, D)),
        _full_spec((D, D)), _full_spec((1, D)),
        _full_spec((D, D)), _full_spec((1, D)),
        _full_spec((D, 2)), _full_spec((1, 2)),
    ],
    out_specs=[_row_spec(), pl.BlockSpec((BS, 2), lambda i: (i, 0)), _row_spec()],
    out_shape=[
        jax.ShapeDtypeStruct((N, D), _F32),
        jax.ShapeDtypeStruct((N, 2), _F32),
        jax.ShapeDtypeStruct((N, D), _F32),
    ],
)

_k_final = pl.pallas_call(
    _k_final_body,
    grid=(NBLK,),
    in_specs=[
        _row_spec(),
        _full_spec((8, D)),
        _full_spec((1, D)), _full_spec((1, D)),
        pl.BlockSpec((1, 1, BS), lambda i: (i, 0, 0)),
        _full_spec((D, D // 2)), _full_spec((1, D // 2)),
        _full_spec((D // 2, 10)), _full_spec((1, 10)),
    ],
    out_specs=_full_spec((NUM_GRAPHS, 10)),
    out_shape=jax.ShapeDtypeStruct((NUM_GRAPHS, 10), _F32),
    scratch_shapes=[pltpu.VMEM((NUM_GRAPHS, D), _F32)],
)


def kernel(x, edge_index, batch, W_in, b_in, Wa, ba, Wm, bm, Wr, br,
           gamma, beta, W1, b1, W2, b2):
  src = edge_index[0].astype(jnp.int32)
  dst = edge_index[1].astype(jnp.int32)
  packed = src * (1 << PSHIFT) + dst
  pad_s = (jnp.arange(PP, dtype=jnp.int32) * 41) % N
  pad_d = N + (jnp.arange(PP, dtype=jnp.int32) % NDISCARD)
  pad_p = pad_s * (1 << PSHIFT) + pad_d
  p3 = jnp.concatenate(
      [packed.reshape(NW, EP), jnp.broadcast_to(pad_p, (NW, PP))],
      axis=1).reshape(NW, CH, CHUNK)
  batch3 = batch.astype(jnp.int32).reshape(NBLK, 1, BS)

  def wa2(i):
    return Wa[i, :, 0].reshape(2, D).transpose(1, 0)

  def bac(i):
    return jnp.concatenate([ba[i], jnp.zeros((1,), _F32)]).reshape(1, 2)

  def row(v):
    return v.reshape(1, -1)

  hm, tbl, hr = _k_in(x, W_in, row(b_in), Wm[0], row(bm[0]),
                      Wr[0], row(br[0]), wa2(0), bac(0))
  for i in range(L):
    tblp = jnp.pad(tbl, ((0, NDISCARD), (0, 0)))
    lo = lax.bitcast_convert_type(tblp[:, 0].astype(jnp.bfloat16),
                                  jnp.uint16).astype(jnp.uint32)
    hi = lax.bitcast_convert_type(tblp[:, 1].astype(jnp.bfloat16),
                                  jnp.uint16).astype(jnp.uint32)
    gtab = lax.bitcast_convert_type(lo | (hi << 16), jnp.int32)
    aggp = _sc_edge_pass(p3, hm, gtab)
    z, st = _k_stats(aggp, hr)
    if i < L - 1:
      hm, tbl, hr = _k_next(z, st, row(gamma[i]), row(beta[i]),
                            Wm[i + 1], row(bm[i + 1]),
                            Wr[i + 1], row(br[i + 1]),
                            wa2(i + 1), bac(i + 1))
  logits = _k_final(z, st, row(gamma[L - 1]), row(beta[L - 1]), batch3,
                    W1, row(b1), W2, row(b2))
  return logits


# speedup vs baseline: 1.2069x; 1.0953x over previous
import functools

import jax
import jax.numpy as jnp
from jax import lax
from jax.experimental import pallas as pl
from jax.experimental.pallas import tpu as pltpu
from jax.experimental.pallas import tpu_sc as plsc

N = 10000
E = 320000
D = 128
L = 4
NUM_GRAPHS = 64

NC = 2
NS = 16
NW = NC * NS
EP = E // NW
CHUNK = 64
CH = 160
EPP = CH * CHUNK
PP = EPP - EP
NDISCARD = 112
NPAD = N + NDISCARD
ROWS_PT = 632
OUT_ROWS = NS * ROWS_PT
PSHIFT = 15
PMASK = (1 << PSHIFT) - 1
NRING = 4

BS = 2000
NBLK = N // BS

_F32 = jnp.float32


def _sc_body(p_hbm, hm_hbm, gtab_hbm, out_hbm,
             pring, sring, dring, gtab, gbuf, rbuf, obuf, agg,
             rsem0, rsem1, ssem0, ssem1, isem0, isem1, isem2, isem3):
  c = lax.axis_index("c")
  s = lax.axis_index("s")
  wid = c * NS + s

  pltpu.sync_copy(gtab_hbm, gtab)

  def zrow(r, carry):
    for k in range(8):
      obuf[0, r, pl.ds(k * 16, 16)] = jnp.zeros((16,), _F32)
    return carry
  lax.fori_loop(0, CHUNK, zrow, 0)
  base = s * ROWS_PT
  for tt in range(ROWS_PT // CHUNK):
    pltpu.sync_copy(obuf.at[0], agg.at[pl.ds(base + tt * CHUNK, CHUNK)])
  rem = ROWS_PT % CHUNK
  pltpu.sync_copy(obuf.at[0, pl.ds(0, rem)],
                  agg.at[pl.ds(base + ROWS_PT - rem, rem)])
  plsc.subcore_barrier()

  rsems = (rsem0, rsem1)
  ssems = (ssem0, ssem1)
  isems = (isem0, isem1, isem2, isem3)

  def start_idx(j, slot):
    pltpu.async_copy(p_hbm.at[wid, j], pring.at[slot], isems[slot])

  def wait_idx(slot):
    pltpu.make_async_copy(p_hbm.at[wid, 0], pring.at[slot], isems[slot]).wait()

  def unpack(slot):
    for q in range(CHUNK // 16):
      pv = pring[slot, pl.ds(q * 16, 16)]
      sring[slot, pl.ds(q * 16, 16)] = lax.shift_right_logical(pv, PSHIFT)
      dring[slot, pl.ds(q * 16, 16)] = lax.bitwise_and(pv, PMASK)

  def start_gather(slot, b):
    pltpu.async_copy(hm_hbm.at[sring.at[slot]], rbuf.at[b], rsems[b])

  def wait_gather(b):
    pltpu.make_async_copy(hm_hbm.at[sring.at[0]], rbuf.at[b], rsems[b]).wait()

  def start_scatter(slot, b):
    pltpu.async_copy(obuf.at[b], agg.at[dring.at[slot]], ssems[b], add=True)

  def wait_scatter(b):
    pltpu.make_async_copy(obuf.at[b], agg.at[dring.at[0]], ssems[b]).wait()

  himask = jnp.full((16,), -65536, jnp.int32)

  def compute(slot, b):
    for q in range(CHUNK // 16):
      sv = sring[slot, pl.ds(q * 16, 16)]
      dv = dring[slot, pl.ds(q * 16, 16)]
      ws = plsc.load_gather(gtab, [sv])
      wd = plsc.load_gather(gtab, [dv])
      asf = plsc.bitcast(lax.shift_left(ws, 16), _F32)
      adf = plsc.bitcast(lax.bitwise_and(wd, himask), _F32)
      a = asf + adf
      gbuf[pl.ds(q * 16, 16)] = 1.0 / (1.0 + jnp.exp(-a))

    eidx = lax.iota(jnp.int32, 16) * 2

    @plsc.parallel_loop(0, CHUNK, 1, unroll=4)
    def _(r):
      gb = plsc.load_gather(gbuf, [jnp.zeros((16,), jnp.int32) + r])
      orow = obuf.at[b, r]
      for k in range(4):
        w = plsc.bitcast(rbuf[b, r, pl.ds(k * 32, 32)], jnp.int32)
        lo = plsc.bitcast(lax.shift_left(w, 16), _F32) * gb
        hi = plsc.bitcast(lax.bitwise_and(w, himask), _F32) * gb
        plsc.store_scatter(orow, [eidx + (k * 32)], lo)
        plsc.store_scatter(orow, [eidx + (k * 32 + 1)], hi)

  for j in range(4):
    start_idx(j, j)
  for j in range(2):
    wait_idx(j)
    unpack(j)
    start_gather(j, j)

  def body(g, carry):
    for b in range(4):
      j = 4 * g + b
      b2 = b % 2
      wait_gather(b2)

      @pl.when(j >= 2)
      def _():
        wait_scatter(b2)

      @pl.when(j + 4 < CH)
      def _():
        start_idx(j + 4, b)

      @pl.when(j + 2 < CH)
      def _():
        wait_idx((b + 2) % 4)

      @pl.when(j + 2 < CH)
      def _():
        unpack((b + 2) % 4)

      compute(b, b2)
      start_scatter(b, b2)

      @pl.when(j + 2 < CH)
      def _():
        start_gather((b + 2) % 4, b2)
    return carry

  lax.fori_loop(0, CH // 4, body, 0)
  wait_scatter(0)
  wait_scatter(1)
  plsc.subcore_barrier()
  pltpu.sync_copy(agg.at[pl.ds(base, ROWS_PT)],
                  out_hbm.at[c, pl.ds(base, ROWS_PT)])


_sc_edge_pass = functools.partial(
    pl.kernel,
    out_type=jax.ShapeDtypeStruct((NC, OUT_ROWS, D), _F32),
    mesh=plsc.VectorSubcoreMesh(core_axis_name="c", subcore_axis_name="s",
                                num_cores=NC, num_subcores=NS),
    scratch_types=[
        pltpu.VMEM((4, CHUNK), jnp.int32),
        pltpu.VMEM((4, CHUNK), jnp.int32),
        pltpu.VMEM((4, CHUNK), jnp.int32),
        pltpu.VMEM((NPAD,), jnp.int32),
        pltpu.VMEM((CHUNK,), _F32),
        pltpu.VMEM((2, CHUNK, D), jnp.bfloat16),
        pltpu.VMEM((2, CHUNK, D), _F32),
        pltpu.VMEM_SHARED((NPAD, D), _F32),
    ] + [pltpu.SemaphoreType.DMA] * 8,
    compiler_params=pltpu.CompilerParams(needs_layout_passes=False,
                                        use_tc_tiling_on_sc=False),
)(_sc_body)


def _produce(h, wm_ref, bm_ref, wr_ref, br_ref, wa_ref, bac_ref,
             hm_ref, tb_ref, hr_ref):
  hm_ref[...] = jnp.dot(h, wm_ref[...], preferred_element_type=_F32) + bm_ref[...]
  hr_ref[...] = jnp.dot(h, wr_ref[...], preferred_element_type=_F32) + br_ref[...]
  tb_ref[...] = jnp.dot(h, wa_ref[...], preferred_element_type=_F32) + bac_ref[...]


def _k_in_body(x_ref, win_ref, bin_ref, wm_ref, bm_ref, wr_ref, br_ref,
               wa_ref, bac_ref, hm_ref, tb_ref, hr_ref):
  h = jnp.maximum(
      jnp.dot(x_ref[...], win_ref[...], preferred_element_type=_F32)
      + bin_ref[...], 0.0)
  _produce(h, wm_ref, bm_ref, wr_ref, br_ref, wa_ref, bac_ref,
           hm_ref, tb_ref, hr_ref)


def _k_stats_body(aggp_ref, hr_ref, z_ref, st_ref):
  i = pl.program_id(0)
  zb = aggp_ref[0] + aggp_ref[1] + hr_ref[...]
  z_ref[...] = zb

  @pl.when(i == 0)
  def _():
    st_ref[...] = jnp.zeros_like(st_ref)

  colsum = jnp.sum(zb, axis=0, keepdims=True)
  colsq = jnp.sum(zb * zb, axis=0, keepdims=True)
  upd = jnp.concatenate([colsum, colsq, jnp.zeros((6, D), _F32)], axis=0)
  st_ref[...] = st_ref[...] + upd


def _bn_relu(z_ref, st_ref, gamma_ref, beta_ref):
  stt = st_ref[...]
  mean = stt[0:1, :] / N
  var = stt[1:2, :] / N - mean * mean
  inv = lax.rsqrt(var + 1e-5)
  return jnp.maximum((z_ref[...] - mean) * (inv * gamma_ref[...])
                     + beta_ref[...], 0.0)


def _k_next_body(z_ref, st_ref, gamma_ref, beta_ref, wm_ref, bm_ref,
                 wr_ref, br_ref, wa_ref, bac_ref, hm_ref, tb_ref, hr_ref):
  h = _bn_relu(z_ref, st_ref, gamma_ref, beta_ref)
  _produce(h, wm_ref, bm_ref, wr_ref, br_ref, wa_ref, bac_ref,
           hm_ref, tb_ref, hr_ref)


def _k_final_body(z_ref, st_ref, gamma_ref, beta_ref, batch_ref,
                  w1_ref, b1_ref, w2_ref, b2_ref, out_ref, pooled_ref):
  i = pl.program_id(0)
  h = _bn_relu(z_ref, st_ref, gamma_ref, beta_ref)
  bb = batch_ref[0]
  gids = lax.broadcasted_iota(jnp.int32, (NUM_GRAPHS, BS), 0)
  onehot = jnp.where(gids == bb, 1.0, 0.0).astype(_F32)

  @pl.when(i == 0)
  def _():
    pooled_ref[...] = jnp.zeros_like(pooled_ref)

  pooled_ref[...] = pooled_ref[...] + jnp.dot(
      onehot, h, preferred_element_type=_F32)

  @pl.when(i == NBLK - 1)
  def _():
    p = pooled_ref[...]
    o1 = jnp.maximum(jnp.dot(p, w1_ref[...], preferred_element_type=_F32)
                     + b1_ref[...], 0.0)
    out_ref[...] = (jnp.dot(o1, w2_ref[...], preferred_element_type=_F32)
                    + b2_ref[...]) * 0.5


def _row_spec():
  return pl.BlockSpec((BS, D), lambda i: (i, 0))


def _full_spec(shape):
  return pl.BlockSpec(shape, lambda i: tuple(0 for _ in shape))


_k_in = pl.pallas_call(
    _k_in_body,
    grid=(NBLK,),
    in_specs=[
        _row_spec(),
        _full_spec((D, D)), _full_spec((1, D)),
        _full_spec((D, D)), _full_spec((1, D)),
        _full_spec((D, D)), _full_spec((1, D)),
        _full_spec((D, 2)), _full_spec((1, 2)),
    ],
    out_specs=[_row_spec(), pl.BlockSpec((BS, 2), lambda i: (i, 0)), _row_spec()],
    out_shape=[
        jax.ShapeDtypeStruct((N, D), _F32),
        jax.ShapeDtypeStruct((N, 2), _F32),
        jax.ShapeDtypeStruct((N, D), _F32),
    ],
)

_k_stats = pl.pallas_call(
    _k_stats_body,
    grid=(NBLK,),
    in_specs=[
        pl.BlockSpec((NC, BS, D), lambda i: (0, i, 0)),
        _row_spec(),
    ],
    out_specs=[_row_spec(), _full_spec((8, D))],
    out_shape=[
        jax.ShapeDtypeStruct((N, D), _F32),
        jax.ShapeDtypeStruct((8, D), _F32),
    ],
)

_k_next = pl.pallas_call(
    _k_next_body,
    grid=(NBLK,),
    in_specs=[
        _row_spec(),
        _full_spec((8, D)),
        _full_spec((1, D)), _full_spec((1, D)),
        _full_spec((D, D)), _full_spec((1, D)),
        _full_spec((D, D)), _full_spec((1, D)),
        _full_spec((D, 2)), _full_spec((1, 2)),
    ],
    out_specs=[_row_spec(), pl.BlockSpec((BS, 2), lambda i: (i, 0)), _row_spec()],
    out_shape=[
        jax.ShapeDtypeStruct((N, D), _F32),
        jax.ShapeDtypeStruct((N, 2), _F32),
        jax.ShapeDtypeStruct((N, D), _F32),
    ],
)

_k_final = pl.pallas_call(
    _k_final_body,
    grid=(NBLK,),
    in_specs=[
        _row_spec(),
        _full_spec((8, D)),
        _full_spec((1, D)), _full_spec((1, D)),
        pl.BlockSpec((1, 1, BS), lambda i: (i, 0, 0)),
        _full_spec((D, D // 2)), _full_spec((1, D // 2)),
        _full_spec((D // 2, 10)), _full_spec((1, 10)),
    ],
    out_specs=_full_spec((NUM_GRAPHS, 10)),
    out_shape=jax.ShapeDtypeStruct((NUM_GRAPHS, 10), _F32),
    scratch_shapes=[pltpu.VMEM((NUM_GRAPHS, D), _F32)],
)


def kernel(x, edge_index, batch, W_in, b_in, Wa, ba, Wm, bm, Wr, br,
           gamma, beta, W1, b1, W2, b2):
  src = edge_index[0].astype(jnp.int32)
  dst = edge_index[1].astype(jnp.int32)
  packed = src * (1 << PSHIFT) + dst
  pad_s = (jnp.arange(PP, dtype=jnp.int32) * 41) % N
  pad_d = N + (jnp.arange(PP, dtype=jnp.int32) % NDISCARD)
  pad_p = pad_s * (1 << PSHIFT) + pad_d
  p3 = jnp.concatenate(
      [packed.reshape(NW, EP), jnp.broadcast_to(pad_p, (NW, PP))],
      axis=1).reshape(NW, CH, CHUNK)
  batch3 = batch.astype(jnp.int32).reshape(NBLK, 1, BS)

  def wa2(i):
    return Wa[i, :, 0].reshape(2, D).transpose(1, 0)

  def bac(i):
    return jnp.concatenate([ba[i], jnp.zeros((1,), _F32)]).reshape(1, 2)

  def row(v):
    return v.reshape(1, -1)

  hm, tbl, hr = _k_in(x, W_in, row(b_in), Wm[0], row(bm[0]),
                      Wr[0], row(br[0]), wa2(0), bac(0))
  for i in range(L):
    tblp = jnp.pad(tbl, ((0, NDISCARD), (0, 0)))
    lo = lax.bitcast_convert_type(tblp[:, 0].astype(jnp.bfloat16),
                                  jnp.uint16).astype(jnp.uint32)
    hi = lax.bitcast_convert_type(tblp[:, 1].astype(jnp.bfloat16),
                                  jnp.uint16).astype(jnp.uint32)
    gtab = lax.bitcast_convert_type(lo | (hi << 16), jnp.int32)
    aggp = _sc_edge_pass(p3, hm.astype(jnp.bfloat16), gtab)
    z, st = _k_stats(aggp, hr)
    if i < L - 1:
      hm, tbl, hr = _k_next(z, st, row(gamma[i]), row(beta[i]),
                            Wm[i + 1], row(bm[i + 1]),
                            Wr[i + 1], row(br[i + 1]),
                            wa2(i + 1), bac(i + 1))
  logits = _k_final(z, st, row(gamma[L - 1]), row(beta[L - 1]), batch3,
                    W1, row(b1), W2, row(b2))
  return logits


# speedup vs baseline: 1.2659x; 1.0489x over previous
import functools

import jax
import jax.numpy as jnp
from jax import lax
from jax.experimental import pallas as pl
from jax.experimental.pallas import tpu as pltpu
from jax.experimental.pallas import tpu_sc as plsc

N = 10000
E = 320000
D = 128
L = 4
NUM_GRAPHS = 64

NC = 2
NS = 16
NW = NC * NS
EP = E // NW
CHUNK = 64
CH = 160
EPP = CH * CHUNK
PP = EPP - EP
NDISCARD = 112
NPAD = N + NDISCARD
ROWS_PT = 632
OUT_ROWS = NS * ROWS_PT
PSHIFT = 15
PMASK = (1 << PSHIFT) - 1
NRING = 4

BS = 2000
NBLK = N // BS

_F32 = jnp.float32


def _sc_body(p_hbm, hm_hbm, gtab_hbm, out_hbm,
             pring, sring, dring, gtab, gbuf, rbuf, obuf, agg,
             rsem0, rsem1, ssem0, ssem1, isem0, isem1, isem2, isem3):
  c = lax.axis_index("c")
  s = lax.axis_index("s")
  wid = c * NS + s

  pltpu.sync_copy(gtab_hbm, gtab)

  def zrow(r, carry):
    for k in range(8):
      obuf[0, r, pl.ds(k * 16, 16)] = jnp.zeros((16,), _F32)
    return carry
  lax.fori_loop(0, CHUNK, zrow, 0)
  base = s * ROWS_PT
  for tt in range(ROWS_PT // CHUNK):
    pltpu.sync_copy(obuf.at[0], agg.at[pl.ds(base + tt * CHUNK, CHUNK)])
  rem = ROWS_PT % CHUNK
  pltpu.sync_copy(obuf.at[0, pl.ds(0, rem)],
                  agg.at[pl.ds(base + ROWS_PT - rem, rem)])
  plsc.subcore_barrier()

  rsems = (rsem0, rsem1)
  ssems = (ssem0, ssem1)
  isems = (isem0, isem1, isem2, isem3)

  def start_idx(j, slot):
    pltpu.async_copy(p_hbm.at[wid, j], pring.at[slot], isems[slot])

  def wait_idx(slot):
    pltpu.make_async_copy(p_hbm.at[wid, 0], pring.at[slot], isems[slot]).wait()

  def unpack(slot):
    for q in range(CHUNK // 16):
      pv = pring[slot, pl.ds(q * 16, 16)]
      sring[slot, pl.ds(q * 16, 16)] = lax.shift_right_logical(pv, PSHIFT)
      dring[slot, pl.ds(q * 16, 16)] = lax.bitwise_and(pv, PMASK)

  def start_gather(slot, b):
    pltpu.async_copy(hm_hbm.at[sring.at[slot]], rbuf.at[b], rsems[b])

  def wait_gather(b):
    pltpu.make_async_copy(hm_hbm.at[sring.at[0]], rbuf.at[b], rsems[b]).wait()

  def start_scatter(slot, b):
    pltpu.async_copy(obuf.at[b], agg.at[dring.at[slot]], ssems[b], add=True)

  def wait_scatter(b):
    pltpu.make_async_copy(obuf.at[b], agg.at[dring.at[0]], ssems[b]).wait()

  himask = jnp.full((16,), -65536, jnp.int32)

  def compute(slot, b):
    for q in range(CHUNK // 16):
      sv = sring[slot, pl.ds(q * 16, 16)]
      dv = dring[slot, pl.ds(q * 16, 16)]
      ws = plsc.load_gather(gtab, [sv])
      wd = plsc.load_gather(gtab, [dv])
      asf = plsc.bitcast(lax.shift_left(ws, 16), _F32)
      adf = plsc.bitcast(lax.bitwise_and(wd, himask), _F32)
      a = asf + adf
      gbuf[pl.ds(q * 16, 16)] = 1.0 / (1.0 + jnp.exp(-a))

    eidx = lax.iota(jnp.int32, 16) * 2

    @plsc.parallel_loop(0, CHUNK, 1, unroll=4)
    def _(r):
      gb = plsc.load_gather(gbuf, [jnp.zeros((16,), jnp.int32) + r])
      orow = obuf.at[b, r]
      for k in range(4):
        w = plsc.bitcast(rbuf[b, r, pl.ds(k * 32, 32)], jnp.int32)
        lo = plsc.bitcast(lax.shift_left(w, 16), _F32) * gb
        hi = plsc.bitcast(lax.bitwise_and(w, himask), _F32) * gb
        plsc.store_scatter(orow, [eidx + (k * 32)], lo)
        plsc.store_scatter(orow, [eidx + (k * 32 + 1)], hi)

  for j in range(4):
    start_idx(j, j)
  for j in range(2):
    wait_idx(j)
    unpack(j)
    start_gather(j, j)

  def body(g, carry):
    for b in range(4):
      j = 4 * g + b
      b2 = b % 2
      wait_gather(b2)

      @pl.when(j >= 2)
      def _():
        wait_scatter(b2)

      @pl.when(j + 4 < CH)
      def _():
        start_idx(j + 4, b)

      @pl.when(j + 2 < CH)
      def _():
        wait_idx((b + 2) % 4)

      @pl.when(j + 2 < CH)
      def _():
        unpack((b + 2) % 4)

      compute(b, b2)
      start_scatter(b, b2)

      @pl.when(j + 2 < CH)
      def _():
        start_gather((b + 2) % 4, b2)
    return carry

  lax.fori_loop(0, CH // 4, body, 0)
  wait_scatter(0)
  wait_scatter(1)
  plsc.subcore_barrier()
  pltpu.sync_copy(agg.at[pl.ds(base, ROWS_PT)],
                  out_hbm.at[c, pl.ds(base, ROWS_PT)])


_sc_edge_pass = functools.partial(
    pl.kernel,
    out_type=jax.ShapeDtypeStruct((NC, OUT_ROWS, D), _F32),
    mesh=plsc.VectorSubcoreMesh(core_axis_name="c", subcore_axis_name="s",
                                num_cores=NC, num_subcores=NS),
    scratch_types=[
        pltpu.VMEM((4, CHUNK), jnp.int32),
        pltpu.VMEM((4, CHUNK), jnp.int32),
        pltpu.VMEM((4, CHUNK), jnp.int32),
        pltpu.VMEM((NPAD,), jnp.int32),
        pltpu.VMEM((CHUNK,), _F32),
        pltpu.VMEM((2, CHUNK, D), jnp.bfloat16),
        pltpu.VMEM((2, CHUNK, D), _F32),
        pltpu.VMEM_SHARED((NPAD, D), _F32),
    ] + [pltpu.SemaphoreType.DMA] * 8,
    compiler_params=pltpu.CompilerParams(needs_layout_passes=False,
                                        use_tc_tiling_on_sc=False),
)(_sc_body)


def _produce(h, wm_ref, bm_ref, wr_ref, br_ref, wa_ref, bac_ref,
             hm_ref, tb_ref, hr_ref):
  hm = jnp.dot(h, wm_ref[...], preferred_element_type=_F32) + bm_ref[...]
  hm_ref[...] = hm.astype(jnp.bfloat16)
  hr_ref[...] = jnp.dot(h, wr_ref[...], preferred_element_type=_F32) + br_ref[...]
  tb_ref[...] = jnp.dot(h, wa_ref[...], preferred_element_type=_F32) + bac_ref[...]


def _k_in_body(x_ref, win_ref, bin_ref, wm_ref, bm_ref, wr_ref, br_ref,
               wa_ref, bac_ref, hm_ref, tb_ref, hr_ref):
  h = jnp.maximum(
      jnp.dot(x_ref[...], win_ref[...], preferred_element_type=_F32)
      + bin_ref[...], 0.0)
  _produce(h, wm_ref, bm_ref, wr_ref, br_ref, wa_ref, bac_ref,
           hm_ref, tb_ref, hr_ref)


def _stats_phase(aggp_ref, hr_ref, z_ref, st_ref, i):
  zb = aggp_ref[0] + aggp_ref[1] + hr_ref[...]
  z_ref[pl.ds(pl.multiple_of(i * BS, BS), BS), :] = zb

  @pl.when(i == 0)
  def _():
    st_ref[...] = jnp.zeros_like(st_ref)

  colsum = jnp.sum(zb, axis=0, keepdims=True)
  colsq = jnp.sum(zb * zb, axis=0, keepdims=True)
  upd = jnp.concatenate([colsum, colsq, jnp.zeros((6, D), _F32)], axis=0)
  st_ref[...] = st_ref[...] + upd


def _bn_relu(zb, st_ref, gamma_ref, beta_ref):
  stt = st_ref[...]
  mean = stt[0:1, :] / N
  var = stt[1:2, :] / N - mean * mean
  inv = lax.rsqrt(var + 1e-5)
  return jnp.maximum((zb - mean) * (inv * gamma_ref[...]) + beta_ref[...], 0.0)


def _k_mid_body(aggp_ref, hr_ref, gamma_ref, beta_ref, wm_ref, bm_ref,
                wr_ref, br_ref, wa_ref, bac_ref, hm_ref, tb_ref, hro_ref,
                z_ref, st_ref):
  ph = pl.program_id(0)
  i = pl.program_id(1)

  @pl.when(ph == 0)
  def _():
    _stats_phase(aggp_ref, hr_ref, z_ref, st_ref, i)

  @pl.when(ph == 1)
  def _():
    zb = z_ref[pl.ds(pl.multiple_of(i * BS, BS), BS), :]
    h = _bn_relu(zb, st_ref, gamma_ref, beta_ref)
    _produce(h, wm_ref, bm_ref, wr_ref, br_ref, wa_ref, bac_ref,
             hm_ref, tb_ref, hro_ref)


def _k_fin_body(aggp_ref, hr_ref, gamma_ref, beta_ref, batch_ref,
                w1_ref, b1_ref, w2_ref, b2_ref, out_ref,
                z_ref, st_ref, pooled_ref):
  ph = pl.program_id(0)
  i = pl.program_id(1)

  @pl.when(ph == 0)
  def _():
    _stats_phase(aggp_ref, hr_ref, z_ref, st_ref, i)

  @pl.when(ph == 1)
  def _():
    zb = z_ref[pl.ds(pl.multiple_of(i * BS, BS), BS), :]
    h = _bn_relu(zb, st_ref, gamma_ref, beta_ref)
    bb = batch_ref[0]
    gids = lax.broadcasted_iota(jnp.int32, (NUM_GRAPHS, BS), 0)
    onehot = jnp.where(gids == bb, 1.0, 0.0).astype(_F32)

    @pl.when(i == 0)
    def _():
      pooled_ref[...] = jnp.zeros_like(pooled_ref)

    pooled_ref[...] = pooled_ref[...] + jnp.dot(
        onehot, h, preferred_element_type=_F32)

    @pl.when(i == NBLK - 1)
    def _():
      pool = pooled_ref[...]
      o1 = jnp.maximum(jnp.dot(pool, w1_ref[...], preferred_element_type=_F32)
                       + b1_ref[...], 0.0)
      out_ref[...] = (jnp.dot(o1, w2_ref[...], preferred_element_type=_F32)
                      + b2_ref[...]) * 0.5


def _row_spec():
  return pl.BlockSpec((BS, D), lambda i: (i, 0))


def _full_spec(shape, ndim=1):
  return pl.BlockSpec(shape, lambda *g: tuple(0 for _ in shape))


def _p0_row(shape=(BS, D)):
  return pl.BlockSpec(shape, lambda p, i: tuple(
      [jnp.where(p == 0, i, 0)] + [0] * (len(shape) - 1)))


def _p1_row(shape=(BS, D)):
  return pl.BlockSpec(shape, lambda p, i: tuple(
      [jnp.where(p == 1, i, 0)] + [0] * (len(shape) - 1)))


def _aggp_spec():
  return pl.BlockSpec((NC, BS, D),
                      lambda p, i: (0, jnp.where(p == 0, i, 0), 0))


_k_in = pl.pallas_call(
    _k_in_body,
    grid=(NBLK,),
    in_specs=[
        _row_spec(),
        _full_spec((D, D)), _full_spec((1, D)),
        _full_spec((D, D)), _full_spec((1, D)),
        _full_spec((D, D)), _full_spec((1, D)),
        _full_spec((D, 2)), _full_spec((1, 2)),
    ],
    out_specs=[_row_spec(), pl.BlockSpec((BS, 2), lambda i: (i, 0)), _row_spec()],
    out_shape=[
        jax.ShapeDtypeStruct((N, D), jnp.bfloat16),
        jax.ShapeDtypeStruct((N, 2), _F32),
        jax.ShapeDtypeStruct((N, D), _F32),
    ],
)

_k_mid = pl.pallas_call(
    _k_mid_body,
    grid=(2, NBLK),
    in_specs=[
        _aggp_spec(),
        _p0_row(),
        _full_spec((1, D)), _full_spec((1, D)),
        _full_spec((D, D)), _full_spec((1, D)),
        _full_spec((D, D)), _full_spec((1, D)),
        _full_spec((D, 2)), _full_spec((1, 2)),
    ],
    out_specs=[_p1_row(), _p1_row((BS, 2)), _p1_row()],
    out_shape=[
        jax.ShapeDtypeStruct((N, D), jnp.bfloat16),
        jax.ShapeDtypeStruct((N, 2), _F32),
        jax.ShapeDtypeStruct((N, D), _F32),
    ],
    scratch_shapes=[pltpu.VMEM((N, D), _F32), pltpu.VMEM((8, D), _F32)],
)

_k_fin = pl.pallas_call(
    _k_fin_body,
    grid=(2, NBLK),
    in_specs=[
        _aggp_spec(),
        _p0_row(),
        _full_spec((1, D)), _full_spec((1, D)),
        pl.BlockSpec((1, 1, BS), lambda p, i: (jnp.where(p == 1, i, 0), 0, 0)),
        _full_spec((D, D // 2)), _full_spec((1, D // 2)),
        _full_spec((D // 2, 10)), _full_spec((1, 10)),
    ],
    out_specs=_full_spec((NUM_GRAPHS, 10)),
    out_shape=jax.ShapeDtypeStruct((NUM_GRAPHS, 10), _F32),
    scratch_shapes=[pltpu.VMEM((N, D), _F32), pltpu.VMEM((8, D), _F32),
                    pltpu.VMEM((NUM_GRAPHS, D), _F32)],
)


def kernel(x, edge_index, batch, W_in, b_in, Wa, ba, Wm, bm, Wr, br,
           gamma, beta, W1, b1, W2, b2):
  src = edge_index[0].astype(jnp.int32)
  dst = edge_index[1].astype(jnp.int32)
  packed = src * (1 << PSHIFT) + dst
  pad_s = (jnp.arange(PP, dtype=jnp.int32) * 41) % N
  pad_d = N + (jnp.arange(PP, dtype=jnp.int32) % NDISCARD)
  pad_p = pad_s * (1 << PSHIFT) + pad_d
  p3 = jnp.concatenate(
      [packed.reshape(NW, EP), jnp.broadcast_to(pad_p, (NW, PP))],
      axis=1).reshape(NW, CH, CHUNK)
  batch3 = batch.astype(jnp.int32).reshape(NBLK, 1, BS)

  def wa2(i):
    return Wa[i, :, 0].reshape(2, D).transpose(1, 0)

  def bac(i):
    return jnp.concatenate([ba[i], jnp.zeros((1,), _F32)]).reshape(1, 2)

  def row(v):
    return v.reshape(1, -1)

  hm, tbl, hr = _k_in(x, W_in, row(b_in), Wm[0], row(bm[0]),
                      Wr[0], row(br[0]), wa2(0), bac(0))
  for i in range(L):
    tblp = jnp.pad(tbl, ((0, NDISCARD), (0, 0)))
    lo = lax.bitcast_convert_type(tblp[:, 0].astype(jnp.bfloat16),
                                  jnp.uint16).astype(jnp.uint32)
    hi = lax.bitcast_convert_type(tblp[:, 1].astype(jnp.bfloat16),
                                  jnp.uint16).astype(jnp.uint32)
    gtab = lax.bitcast_convert_type(lo | (hi << 16), jnp.int32)
    aggp = _sc_edge_pass(p3, hm, gtab)
    if i < L - 1:
      hm, tbl, hr = _k_mid(aggp, hr, row(gamma[i]), row(beta[i]),
                           Wm[i + 1], row(bm[i + 1]),
                           Wr[i + 1], row(br[i + 1]),
                           wa2(i + 1), bac(i + 1))
  logits = _k_fin(aggp, hr, row(gamma[L - 1]), row(beta[L - 1]), batch3,
                  W1, row(b1), W2, row(b2))
  return logits
